# baseline (speedup 1.0000x reference)
import functools

import jax
import jax.numpy as jnp
from jax import lax
from jax.experimental import pallas as pl
from jax.experimental.pallas import tpu as pltpu
from jax.experimental.pallas import tpu_sc as plsc

N_NODES = 10000
N_EDGES = 320000
D = 128

NC = 2
NS = 16
NW = NC * NS
GB = 128
E_PER_TILE = 10240
E_PAD = NW * E_PER_TILE
STEPS = E_PER_TILE // GB
N_ACC = 10112
ROWS_PER_TILE = N_ACC // NS

_mesh = lambda: plsc.VectorSubcoreMesh(core_axis_name="c", subcore_axis_name="s")


DEG_R = 40
DEG_C = 256


@functools.partial(
    pl.kernel,
    mesh=_mesh(),
    out_type=jax.ShapeDtypeStruct((NW, DEG_R, DEG_C), jnp.float32),
    scratch_types=[
        pltpu.VMEM((E_PER_TILE,), jnp.int32),
        pltpu.VMEM((DEG_R, DEG_C), jnp.float32),
    ],
    compiler_params=pltpu.CompilerParams(needs_layout_passes=False),
)
def _deg_sc(dst_hbm, out_hbm, dst_v, deg_v):
    c = lax.axis_index("c")
    s = lax.axis_index("s")
    wid = c * NS + s
    pltpu.sync_copy(dst_hbm.at[pl.ds(wid * E_PER_TILE, E_PER_TILE)], dst_v)

    zeros16 = jnp.zeros((16,), jnp.float32)

    def zbody(i, _):
        deg_v[lax.div(i, 16), pl.ds(lax.rem(i, 16) * 16, 16)] = zeros16
        return 0

    lax.fori_loop(0, DEG_R * 16, zbody, 0)

    ones16 = jnp.ones((16,), jnp.float32)

    def body(r, _):
        for k in range(4):
            d = dst_v[pl.ds(r * 64 + k * 16, 16)]
            row = lax.shift_right_logical(d, 8)
            col = lax.bitwise_and(d, 255)
            plsc.addupdate_scatter(deg_v, [row, col], ones16)
        return 0

    lax.fori_loop(0, E_PER_TILE // 64, body, 0)
    pltpu.sync_copy(deg_v, out_hbm.at[wid])


NSLOT = 3


@functools.partial(
    pl.kernel,
    mesh=_mesh(),
    out_type=jax.ShapeDtypeStruct((NC, NS, ROWS_PER_TILE, D), jnp.float32),
    scratch_types=[
        pltpu.VMEM((NSLOT, GB), jnp.int32),
        pltpu.VMEM((NSLOT, GB), jnp.int32),
        pltpu.VMEM((NSLOT, GB, D), jnp.float32),
        pltpu.VMEM_SHARED((N_ACC, D), jnp.float32),
        pltpu.SemaphoreType.DMA((NSLOT,)),
        pltpu.SemaphoreType.DMA((NSLOT,)),
    ],
    compiler_params=pltpu.CompilerParams(needs_layout_passes=False),
)
def _scatter_sc(h_hbm, src_hbm, dst_hbm, out_hbm,
                sidx_v, didx_v, rows_v, acc_sh, isem, gsem):
    c = lax.axis_index("c")
    s = lax.axis_index("s")
    wid = c * NS + s
    base = wid * E_PER_TILE

    zeros16 = jnp.zeros((16,), jnp.float32)

    def zbody(i, _):
        rows_v[0, lax.div(i, 8), pl.ds(lax.rem(i, 8) * 16, 16)] = zeros16
        return 0

    lax.fori_loop(0, GB * 8, zbody, 0)
    for k in range(4):
        pltpu.async_copy(
            rows_v.at[0], acc_sh.at[pl.ds(s * ROWS_PER_TILE + k * GB, GB)],
            isem.at[0])
    pltpu.async_copy(
        rows_v.at[0, pl.ds(0, ROWS_PER_TILE - 4 * GB)],
        acc_sh.at[pl.ds(s * ROWS_PER_TILE + 4 * GB, ROWS_PER_TILE - 4 * GB)],
        isem.at[0])
    for k in range(4):
        pltpu.make_async_copy(
            rows_v.at[0], acc_sh.at[pl.ds(s * ROWS_PER_TILE + k * GB, GB)],
            isem.at[0]).wait()
    pltpu.make_async_copy(
        rows_v.at[0, pl.ds(0, ROWS_PER_TILE - 4 * GB)],
        acc_sh.at[pl.ds(s * ROWS_PER_TILE + 4 * GB, ROWS_PER_TILE - 4 * GB)],
        isem.at[0]).wait()

    def istart(j, slot):
        pltpu.async_copy(src_hbm.at[pl.ds(base + j * GB, GB)], sidx_v.at[slot],
                         isem.at[slot])
        pltpu.async_copy(dst_hbm.at[pl.ds(base + j * GB, GB)], didx_v.at[slot],
                         isem.at[slot])

    def iwait(j, slot):
        pltpu.make_async_copy(
            src_hbm.at[pl.ds(base + j * GB, GB)], sidx_v.at[slot],
            isem.at[slot]).wait()
        pltpu.make_async_copy(
            dst_hbm.at[pl.ds(base + j * GB, GB)], didx_v.at[slot],
            isem.at[slot]).wait()

    def gstart(j, slot):
        pltpu.async_copy(h_hbm.at[sidx_v.at[slot]], rows_v.at[slot],
                         gsem.at[slot])

    def gwait(j, slot):
        pltpu.make_async_copy(
            h_hbm.at[sidx_v.at[slot]], rows_v.at[slot], gsem.at[slot]
        ).wait()

    def scat(j, slot):
        pltpu.sync_copy(rows_v.at[slot], acc_sh.at[didx_v.at[slot]],
                        add=True)

    plsc.subcore_barrier()

    for k in range(NSLOT):
        istart(k, k)
    for k in range(2):
        iwait(k, k)
        gstart(k, k)

    def step(j, _):
        slot = lax.rem(j, NSLOT)
        gwait(j, slot)
        scat(j, slot)

        @pl.when(j + NSLOT < STEPS)
        def _():
            istart(j + NSLOT, slot)

        @pl.when(j + 2 < STEPS)
        def _():
            slot2 = lax.rem(j + 2, NSLOT)
            iwait(j + 2, slot2)
            gstart(j + 2, slot2)

        return 0

    lax.fori_loop(0, STEPS, step, 0)
    plsc.subcore_barrier()
    pltpu.sync_copy(
        acc_sh.at[pl.ds(s * ROWS_PER_TILE, ROWS_PER_TILE)],
        out_hbm.at[c, s],
    )


RB = 1000
NBLK = N_NODES // RB


def _tc1_body(degp_ref, x_ref, w_ref, dinv_ref, h1p_ref):
    dinv = lax.rsqrt(degp_ref[...] + 1.0)
    h = jnp.dot(x_ref[...], w_ref[...], preferred_element_type=jnp.float32)
    dinv_ref[...] = dinv
    h1p_ref[...] = h * dinv


def _tc1(deg_col, x, W1):
    return pl.pallas_call(
        _tc1_body,
        grid=(NBLK,),
        in_specs=[
            pl.BlockSpec((RB, 1), lambda i: (i, 0)),
            pl.BlockSpec((RB, D), lambda i: (i, 0)),
            pl.BlockSpec((D, D), lambda i: (0, 0)),
        ],
        out_specs=[
            pl.BlockSpec((RB, 1), lambda i: (i, 0)),
            pl.BlockSpec((RB, D), lambda i: (i, 0)),
        ],
        out_shape=[
            jax.ShapeDtypeStruct((N_NODES, 1), jnp.float32),
            jax.ShapeDtypeStruct((N_NODES, D), jnp.float32),
        ],
    )(deg_col, x, W1)


def _tc2_body(acc_ref, hp_ref, dinv_ref, b_ref, w_ref, out_ref):
    dinv = dinv_ref[...]
    z = dinv * (acc_ref[0] + acc_ref[1] + hp_ref[...]) + b_ref[...]
    z = jnp.maximum(z, 0.0)
    h = jnp.dot(z, w_ref[...], preferred_element_type=jnp.float32)
    out_ref[...] = h * dinv


def _tc2(acc, hp, dinv, b, W2):
    return pl.pallas_call(
        _tc2_body,
        grid=(NBLK,),
        in_specs=[
            pl.BlockSpec((NC, RB, D), lambda i: (0, i, 0)),
            pl.BlockSpec((RB, D), lambda i: (i, 0)),
            pl.BlockSpec((RB, 1), lambda i: (i, 0)),
            pl.BlockSpec((1, D), lambda i: (0, 0)),
            pl.BlockSpec((D, D), lambda i: (0, 0)),
        ],
        out_specs=pl.BlockSpec((RB, D), lambda i: (i, 0)),
        out_shape=jax.ShapeDtypeStruct((N_NODES, D), jnp.float32),
    )(acc, hp, dinv, b, W2)


def _tc3_body(acc_ref, hp_ref, dinv_ref, b_ref, wl_ref, bl_ref, out_ref):
    dinv = dinv_ref[...]
    z = dinv * (acc_ref[0] + acc_ref[1] + hp_ref[...]) + b_ref[...]
    z = jnp.maximum(z, 0.0)
    out = lax.dot_general(z, wl_ref[...], (((1,), (1,)), ((), ())),
                          preferred_element_type=jnp.float32)
    out_ref[...] = out + bl_ref[...]


def _tc3(acc, hp, dinv, b, Wl, bl):
    ncls = Wl.shape[0]
    return pl.pallas_call(
        _tc3_body,
        grid=(NBLK,),
        in_specs=[
            pl.BlockSpec((NC, RB, D), lambda i: (0, i, 0)),
            pl.BlockSpec((RB, D), lambda i: (i, 0)),
            pl.BlockSpec((RB, 1), lambda i: (i, 0)),
            pl.BlockSpec((1, D), lambda i: (0, 0)),
            pl.BlockSpec((ncls, D), lambda i: (0, 0)),
            pl.BlockSpec((1, ncls), lambda i: (0, 0)),
        ],
        out_specs=pl.BlockSpec((RB, ncls), lambda i: (i, 0)),
        out_shape=jax.ShapeDtypeStruct((N_NODES, ncls), jnp.float32),
    )(acc, hp, dinv, b, Wl, bl)


def kernel(x, edge_index, W1, b1, W2, b2, Wl, bl):
    npad = E_PAD - N_EDGES
    pad_src = jnp.arange(npad, dtype=jnp.int32)
    pad_dst = N_NODES + (pad_src % (N_ACC - N_NODES))
    src_p = jnp.concatenate([edge_index[0], pad_src])
    dst_p = jnp.concatenate([edge_index[1], pad_dst])

    deg_parts = _deg_sc(dst_p)
    deg_col = deg_parts.sum(axis=0).reshape(DEG_R * DEG_C)[:N_NODES]
    dinv, h1p = _tc1(deg_col.reshape(N_NODES, 1), x, W1)
    acc1 = _scatter_sc(h1p, src_p, dst_p).reshape(NC, N_ACC, D)
    h2p = _tc2(acc1, h1p, dinv, b1.reshape(1, D), W2)
    acc2 = _scatter_sc(h2p, src_p, dst_p).reshape(NC, N_ACC, D)
    out = _tc3(acc2, h2p, dinv, b2.reshape(1, D), Wl, bl.reshape(1, Wl.shape[0]))
    return out


# speedup vs baseline: 1.0919x; 1.0919x over previous
import functools

import jax
import jax.numpy as jnp
from jax import lax
from jax.experimental import pallas as pl
from jax.experimental.pallas import tpu as pltpu
from jax.experimental.pallas import tpu_sc as plsc

N_NODES = 10000
N_EDGES = 320000
D = 128

NC = 2
NS = 16
NW = NC * NS
GB = 80
E_PER_TILE = N_EDGES // NW
STEPS = E_PER_TILE // GB
N_ACC = 10240
ROWS_PER_TILE = N_ACC // NS

_mesh = lambda: plsc.VectorSubcoreMesh(core_axis_name="c", subcore_axis_name="s")


DEG_R = 40
DEG_C = 256


@functools.partial(
    pl.kernel,
    mesh=_mesh(),
    out_type=jax.ShapeDtypeStruct((NW, DEG_R, DEG_C), jnp.float32),
    scratch_types=[
        pltpu.VMEM((STEPS, 1, GB), jnp.int32),
        pltpu.VMEM((DEG_R, DEG_C), jnp.float32),
    ],
    compiler_params=pltpu.CompilerParams(needs_layout_passes=False),
)
def _deg_sc(idx_hbm, out_hbm, dst_v, deg_v):
    c = lax.axis_index("c")
    s = lax.axis_index("s")
    wid = c * NS + s
    pltpu.sync_copy(idx_hbm.at[1, wid], dst_v)

    zeros16 = jnp.zeros((16,), jnp.float32)

    def zbody(i, _):
        deg_v[lax.div(i, 16), pl.ds(lax.rem(i, 16) * 16, 16)] = zeros16
        return 0

    lax.fori_loop(0, DEG_R * 16, zbody, 0)

    ones16 = jnp.ones((16,), jnp.float32)

    def body(r, _):
        for k in range(GB // 16):
            d = dst_v[r, 0, pl.ds(k * 16, 16)]
            row = lax.shift_right_logical(d, 8)
            col = lax.bitwise_and(d, 255)
            plsc.addupdate_scatter(deg_v, [row, col], ones16)
        return 0

    lax.fori_loop(0, STEPS, body, 0)
    pltpu.sync_copy(deg_v, out_hbm.at[wid])


NSLOT = 4


@functools.partial(
    pl.kernel,
    mesh=_mesh(),
    out_type=jax.ShapeDtypeStruct((NC, NS, ROWS_PER_TILE, D), jnp.float32),
    scratch_types=[
        pltpu.VMEM((NSLOT, 2, 1, GB), jnp.int32),
        pltpu.VMEM((NSLOT, GB, D), jnp.float32),
        pltpu.VMEM_SHARED((N_ACC, D), jnp.float32),
        pltpu.SemaphoreType.DMA((NSLOT,)),
        pltpu.SemaphoreType.DMA((NSLOT,)),
    ],
    compiler_params=pltpu.CompilerParams(needs_layout_passes=False),
)
def _scatter_sc(h_hbm, idx_hbm, out_hbm, idx_v, rows_v, acc_sh, isem, gsem):
    c = lax.axis_index("c")
    s = lax.axis_index("s")
    wid = c * NS + s

    zeros16 = jnp.zeros((16,), jnp.float32)

    def zbody(i, _):
        rows_v[0, lax.div(i, 8), pl.ds(lax.rem(i, 8) * 16, 16)] = zeros16
        return 0

    lax.fori_loop(0, GB * 8, zbody, 0)
    for k in range(ROWS_PER_TILE // GB):
        pltpu.async_copy(
            rows_v.at[0], acc_sh.at[pl.ds(s * ROWS_PER_TILE + k * GB, GB)],
            isem.at[0])
    for k in range(ROWS_PER_TILE // GB):
        pltpu.make_async_copy(
            rows_v.at[0], acc_sh.at[pl.ds(s * ROWS_PER_TILE + k * GB, GB)],
            isem.at[0]).wait()

    def istart(j, slot):
        pltpu.async_copy(idx_hbm.at[:, wid, j], idx_v.at[slot], isem.at[slot])

    def iwait(j, slot):
        pltpu.make_async_copy(
            idx_hbm.at[:, wid, j], idx_v.at[slot], isem.at[slot]
        ).wait()

    def gstart(j, slot):
        pltpu.async_copy(h_hbm.at[idx_v.at[slot, 0, 0]], rows_v.at[slot],
                         gsem.at[slot])

    def gwait(j, slot):
        pltpu.make_async_copy(
            h_hbm.at[idx_v.at[slot, 0, 0]], rows_v.at[slot], gsem.at[slot]
        ).wait()

    def scat(j, slot):
        pltpu.sync_copy(rows_v.at[slot], acc_sh.at[idx_v.at[slot, 1, 0]],
                        add=True)

    plsc.subcore_barrier()

    for k in range(NSLOT):
        istart(k, k)
    for k in range(3):
        iwait(k, k)
        gstart(k, k)

    def step(j, _):
        slot = lax.rem(j, NSLOT)
        gwait(j, slot)
        scat(j, slot)

        @pl.when(j + NSLOT < STEPS)
        def _():
            istart(j + NSLOT, slot)

        @pl.when(j + 3 < STEPS)
        def _():
            slot3 = lax.rem(j + 3, NSLOT)
            iwait(j + 3, slot3)
            gstart(j + 3, slot3)

        return 0

    lax.fori_loop(0, STEPS, step, 0)
    plsc.subcore_barrier()
    pltpu.sync_copy(
        acc_sh.at[pl.ds(s * ROWS_PER_TILE, ROWS_PER_TILE)],
        out_hbm.at[c, s],
    )


RB = 1000
NBLK = N_NODES // RB


def _tc1_body(degp_ref, x_ref, w_ref, dinv_ref, h1p_ref):
    dinv = lax.rsqrt(degp_ref[...] + 1.0)
    h = jnp.dot(x_ref[...], w_ref[...], preferred_element_type=jnp.float32)
    dinv_ref[...] = dinv
    h1p_ref[...] = h * dinv


def _tc1(deg_col, x, W1):
    return pl.pallas_call(
        _tc1_body,
        grid=(NBLK,),
        in_specs=[
            pl.BlockSpec((RB, 1), lambda i: (i, 0)),
            pl.BlockSpec((RB, D), lambda i: (i, 0)),
            pl.BlockSpec((D, D), lambda i: (0, 0)),
        ],
        out_specs=[
            pl.BlockSpec((RB, 1), lambda i: (i, 0)),
            pl.BlockSpec((RB, D), lambda i: (i, 0)),
        ],
        out_shape=[
            jax.ShapeDtypeStruct((N_NODES, 1), jnp.float32),
            jax.ShapeDtypeStruct((N_NODES, D), jnp.float32),
        ],
    )(deg_col, x, W1)


def _tc2_body(acc_ref, hp_ref, dinv_ref, b_ref, w_ref, out_ref):
    dinv = dinv_ref[...]
    z = dinv * (acc_ref[0] + acc_ref[1] + hp_ref[...]) + b_ref[...]
    z = jnp.maximum(z, 0.0)
    h = jnp.dot(z, w_ref[...], preferred_element_type=jnp.float32)
    out_ref[...] = h * dinv


def _tc2(acc, hp, dinv, b, W2):
    return pl.pallas_call(
        _tc2_body,
        grid=(NBLK,),
        in_specs=[
            pl.BlockSpec((NC, RB, D), lambda i: (0, i, 0)),
            pl.BlockSpec((RB, D), lambda i: (i, 0)),
            pl.BlockSpec((RB, 1), lambda i: (i, 0)),
            pl.BlockSpec((1, D), lambda i: (0, 0)),
            pl.BlockSpec((D, D), lambda i: (0, 0)),
        ],
        out_specs=pl.BlockSpec((RB, D), lambda i: (i, 0)),
        out_shape=jax.ShapeDtypeStruct((N_NODES, D), jnp.float32),
    )(acc, hp, dinv, b, W2)


def _tc3_body(acc_ref, hp_ref, dinv_ref, b_ref, wl_ref, bl_ref, out_ref):
    dinv = dinv_ref[...]
    z = dinv * (acc_ref[0] + acc_ref[1] + hp_ref[...]) + b_ref[...]
    z = jnp.maximum(z, 0.0)
    out = lax.dot_general(z, wl_ref[...], (((1,), (1,)), ((), ())),
                          preferred_element_type=jnp.float32)
    out_ref[...] = out + bl_ref[...]


def _tc3(acc, hp, dinv, b, Wl, bl):
    ncls = Wl.shape[0]
    return pl.pallas_call(
        _tc3_body,
        grid=(NBLK,),
        in_specs=[
            pl.BlockSpec((NC, RB, D), lambda i: (0, i, 0)),
            pl.BlockSpec((RB, D), lambda i: (i, 0)),
            pl.BlockSpec((RB, 1), lambda i: (i, 0)),
            pl.BlockSpec((1, D), lambda i: (0, 0)),
            pl.BlockSpec((ncls, D), lambda i: (0, 0)),
            pl.BlockSpec((1, ncls), lambda i: (0, 0)),
        ],
        out_specs=pl.BlockSpec((RB, ncls), lambda i: (i, 0)),
        out_shape=jax.ShapeDtypeStruct((N_NODES, ncls), jnp.float32),
    )(acc, hp, dinv, b, Wl, bl)


def kernel(x, edge_index, W1, b1, W2, b2, Wl, bl):
    idx = edge_index.reshape(2, NW, STEPS, 1, GB)

    deg_parts = _deg_sc(idx)
    deg_col = deg_parts.sum(axis=0).reshape(DEG_R * DEG_C)[:N_NODES]
    dinv, h1p = _tc1(deg_col.reshape(N_NODES, 1), x, W1)
    acc1 = _scatter_sc(h1p, idx).reshape(NC, N_ACC, D)
    h2p = _tc2(acc1, h1p, dinv, b1.reshape(1, D), W2)
    acc2 = _scatter_sc(h2p, idx).reshape(NC, N_ACC, D)
    out = _tc3(acc2, h2p, dinv, b2.reshape(1, D), Wl, bl.reshape(1, Wl.shape[0]))
    return out
